# baseline (speedup 1.0000x reference)
import jax
import jax.numpy as jnp
from jax.experimental import pallas as pl
from jax.experimental.pallas import tpu as pltpu

_B, _S, _HID = 2, 2048, 2048
_INTER = 5504
_RET = 128
_NE = 64
_TOPK = 8
_NK = 8

_INTER_PAD = 5632
_TN = 1024
_TK = 512


def _repack_kernel(wg_ref, wu_ref, wd_ref, h_ref,
                   wgo_ref, wuo_ref, wdo_ref, ho_ref):
    i = pl.program_id(0)
    nchunk = pl.num_programs(0)
    wg = wg_ref[...].astype(jnp.bfloat16)
    wu = wu_ref[...].astype(jnp.bfloat16)
    wd = wd_ref[...].astype(jnp.bfloat16)

    @pl.when(i < nchunk - 1)
    def _():
        wgo_ref[...] = wg
        wuo_ref[...] = wu
        wdo_ref[...] = wd

    @pl.when(i == nchunk - 1)
    def _():
        valid = _INTER % wg.shape[0]
        rr = jax.lax.broadcasted_iota(jnp.int32, wg.shape, 0)
        zb = jnp.zeros((), jnp.bfloat16)
        wgo_ref[...] = jnp.where(rr < valid, wg, zb)
        wuo_ref[...] = jnp.where(rr < valid, wu, zb)
        cc = jax.lax.broadcasted_iota(jnp.int32, wd.shape, 1)
        wdo_ref[...] = jnp.where(cc < valid, wd, zb)

    ho_ref[...] = h_ref[...].astype(jnp.bfloat16)


def _moe_mlp_kernel(h0_ref, h1_ref, hflat_ref, wq_ref, keys_ref,
                    down_ref, up_ref, wg_ref, wu_ref, wd_ref, out_ref,
                    a_ref):
    k = pl.program_id(1)
    kt = pl.num_programs(1) - 1
    f32 = jnp.float32
    dnT = (((1,), (1,)), ((), ()))

    @pl.when(k == 0)
    def _routing():
        hb = hflat_ref[...]

        dn0 = (((0,), (0,)), ((), ()))
        wq = wq_ref[...]
        k0 = keys_ref[0:64, :]
        k1 = keys_ref[64:128, :]
        p0a = jax.lax.dot_general(wq[0:64, :], k0, dn0, preferred_element_type=f32)
        p0b = jax.lax.dot_general(wq[64:128, :], k0, dn0, preferred_element_type=f32)
        p1a = jax.lax.dot_general(wq[0:64, :], k1, dn0, preferred_element_type=f32)
        p1b = jax.lax.dot_general(wq[64:128, :], k1, dn0, preferred_element_type=f32)
        h0 = h0_ref[...]
        h1 = h1_ref[...]
        a0a = jnp.dot(h0, p0a, preferred_element_type=f32)
        a0b = jnp.dot(h0, p0b, preferred_element_type=f32)
        a1a = jnp.dot(h1, p1a, preferred_element_type=f32)
        a1b = jnp.dot(h1, p1b, preferred_element_type=f32)

        col = jax.lax.broadcasted_iota(jnp.int32, (8, 64), 1)
        row = jax.lax.broadcasted_iota(jnp.int32, (8, 64), 0)
        e1 = (col // 8 == row).astype(f32)
        e2 = (col % 8 == row).astype(f32)

        def _masked_softmax_top8(s64):
            cur = s64
            m0 = jnp.max(cur, axis=1, keepdims=True)
            m = m0
            for _ in range(_TOPK - 1):
                cur = jnp.where(cur >= m, -jnp.inf, cur)
                m = jnp.max(cur, axis=1, keepdims=True)
            p = jnp.where(s64 >= m, jnp.exp(s64 - m0), 0.0)
            return p / jnp.sum(p, axis=1, keepdims=True)

        s64e = (jnp.dot(a0a, e1, preferred_element_type=f32)
                + jnp.dot(a1a, e2, preferred_element_type=f32))
        s64o = (jnp.dot(a0b, e1, preferred_element_type=f32)
                + jnp.dot(a1b, e2, preferred_element_type=f32))
        pe = _masked_softmax_top8(s64e)
        po = _masked_softmax_top8(s64o)

        rr = jax.lax.broadcasted_iota(jnp.int32, (_TN, _TN // 2), 0)
        cc = jax.lax.broadcasted_iota(jnp.int32, (_TN, _TN // 2), 1)
        ea = (rr == 2 * cc).astype(f32)
        eb = (rr == 2 * cc + 1).astype(f32)
        p = (jnp.dot(ea, pe, preferred_element_type=f32)
             + jnp.dot(eb, po, preferred_element_type=f32))

        L = jax.lax.dot_general(hb, down_ref[...].astype(jnp.bfloat16),
                                (((1,), (1,)), ((), ())),
                                preferred_element_type=f32)
        w64 = L * jax.nn.sigmoid(L) * p
        out_ref[...] = jnp.dot(w64, up_ref[...], preferred_element_type=f32)

        g = jax.lax.dot_general(hb, wg_ref[...], dnT, preferred_element_type=f32)
        u = jax.lax.dot_general(hb, wu_ref[...], dnT, preferred_element_type=f32)
        a_ref[pl.ds(0, 1)] = (g * jax.nn.sigmoid(g) * u).astype(jnp.bfloat16)[None]

    @pl.when(jnp.logical_and(k > 0, k < kt))
    def _steady():
        hb = hflat_ref[...]
        rd = jax.lax.rem(k + 1, 2)
        wr = jax.lax.rem(k, 2)
        ap = a_ref[pl.ds(rd, 1)][0]
        wd = wd_ref[...]
        out_ref[:, 0:_HID // 2] += jax.lax.dot_general(
            ap, wd[0:_HID // 2, :], dnT, preferred_element_type=f32)
        out_ref[:, _HID // 2:] += jax.lax.dot_general(
            ap, wd[_HID // 2:, :], dnT, preferred_element_type=f32)
        g = jax.lax.dot_general(hb, wg_ref[...], dnT, preferred_element_type=f32)
        u = jax.lax.dot_general(hb, wu_ref[...], dnT, preferred_element_type=f32)
        a_ref[pl.ds(wr, 1)] = (g * jax.nn.sigmoid(g) * u).astype(jnp.bfloat16)[None]

    @pl.when(k == kt)
    def _drain():
        out_ref[...] += jax.lax.dot_general(
            a_ref[pl.ds((kt - 1) % 2, 1)][0], wd_ref[...], dnT,
            preferred_element_type=f32)


def kernel(hidden_states, Wq, keys, down_embed, up_embed, Wg, Wu, Wd):
    b, s, h = hidden_states.shape
    N = b * s
    hflat_f = hidden_states.reshape(N, h)
    keys2 = keys.reshape(2 * (_RET // 2), _NK)

    rp = 256
    nchunk = _INTER_PAD // rp
    wg_p, wu_p, wd_p, hflat = pl.pallas_call(
        _repack_kernel,
        grid=(nchunk,),
        in_specs=[
            pl.BlockSpec((rp, h), lambda i: (i, 0)),
            pl.BlockSpec((rp, h), lambda i: (i, 0)),
            pl.BlockSpec((h, rp), lambda i: (0, i)),
            pl.BlockSpec((rp, h), lambda i: (jnp.minimum(i, N // rp - 1), 0)),
        ],
        out_specs=[
            pl.BlockSpec((rp, h), lambda i: (i, 0)),
            pl.BlockSpec((rp, h), lambda i: (i, 0)),
            pl.BlockSpec((h, rp), lambda i: (0, i)),
            pl.BlockSpec((rp, h), lambda i: (jnp.minimum(i, N // rp - 1), 0)),
        ],
        out_shape=[
            jax.ShapeDtypeStruct((_INTER_PAD, h), jnp.bfloat16),
            jax.ShapeDtypeStruct((_INTER_PAD, h), jnp.bfloat16),
            jax.ShapeDtypeStruct((h, _INTER_PAD), jnp.bfloat16),
            jax.ShapeDtypeStruct((N, h), jnp.bfloat16),
        ],
    )(Wg, Wu, Wd, hflat_f)

    nt = N // _TN
    kt = _INTER_PAD // _TK
    out = pl.pallas_call(
        _moe_mlp_kernel,
        grid=(nt, kt + 1),
        in_specs=[
            pl.BlockSpec((_TN // 2, h), lambda n, k: (n, 0)),
            pl.BlockSpec((_TN // 2, h), lambda n, k: (n + _S // (_TN // 2), 0)),
            pl.BlockSpec((_TN, h), lambda n, k: (n, 0)),
            pl.BlockSpec((_RET, h), lambda n, k: (0, 0)),
            pl.BlockSpec((2 * (_RET // 2), _NK), lambda n, k: (0, 0)),
            pl.BlockSpec((_NE, h), lambda n, k: (0, 0)),
            pl.BlockSpec((_NE, h), lambda n, k: (0, 0)),
            pl.BlockSpec((_TK, h), lambda n, k: (jnp.minimum(k, kt - 1), 0)),
            pl.BlockSpec((_TK, h), lambda n, k: (jnp.minimum(k, kt - 1), 0)),
            pl.BlockSpec((h, _TK), lambda n, k: (0, jnp.maximum(k - 1, 0))),
        ],
        out_specs=pl.BlockSpec((_TN, h), lambda n, k: (n, 0)),
        out_shape=jax.ShapeDtypeStruct((N, h), jnp.float32),
        scratch_shapes=[pltpu.VMEM((2, _TN, _TK), jnp.bfloat16)],
        compiler_params=pltpu.CompilerParams(
            dimension_semantics=("arbitrary", "arbitrary"),
        ),
    )(hflat, hflat, hflat, Wq, keys2,
      down_embed, up_embed, wg_p, wu_p, wd_p)

    return out.reshape(b, s, h)


# speedup vs baseline: 1.0010x; 1.0010x over previous
import jax
import jax.numpy as jnp
from jax.experimental import pallas as pl
from jax.experimental.pallas import tpu as pltpu

_B, _S, _HID = 2, 2048, 2048
_INTER = 5504
_RET = 128
_NE = 64
_TOPK = 8
_NK = 8

_INTER_PAD = 5632
_TN = 1024
_TK = 512


def _repack_kernel(wg_ref, wu_ref, wd_ref, h_ref,
                   wgo_ref, wuo_ref, wdo_ref, ho_ref):
    i = pl.program_id(0)
    nchunk = pl.num_programs(0)
    wg = wg_ref[...].astype(jnp.bfloat16)
    wu = wu_ref[...].astype(jnp.bfloat16)
    wd = wd_ref[...].astype(jnp.bfloat16)

    @pl.when(i < nchunk - 1)
    def _():
        wgo_ref[...] = wg
        wuo_ref[...] = wu
        wdo_ref[...] = wd

    @pl.when(i == nchunk - 1)
    def _():
        valid = _INTER % wg.shape[0]
        rr = jax.lax.broadcasted_iota(jnp.int32, wg.shape, 0)
        zb = jnp.zeros((), jnp.bfloat16)
        wgo_ref[...] = jnp.where(rr < valid, wg, zb)
        wuo_ref[...] = jnp.where(rr < valid, wu, zb)
        cc = jax.lax.broadcasted_iota(jnp.int32, wd.shape, 1)
        wdo_ref[...] = jnp.where(cc < valid, wd, zb)

    ho_ref[...] = h_ref[...].astype(jnp.bfloat16)


def _moe_mlp_kernel(h0_ref, h1_ref, hflat_ref, wq_ref, keys_ref,
                    down_ref, up_ref, wg_ref, wu_ref, wd_ref, out_ref,
                    a_ref):
    k = pl.program_id(1)
    kt = pl.num_programs(1) - 1
    f32 = jnp.float32
    dnT = (((1,), (1,)), ((), ()))

    @pl.when(k == 0)
    def _routing():
        hb = hflat_ref[...]

        dn0 = (((0,), (0,)), ((), ()))
        wq = wq_ref[...]
        k0 = keys_ref[0:64, :]
        k1 = keys_ref[64:128, :]
        p0a = jax.lax.dot_general(wq[0:64, :], k0, dn0, preferred_element_type=f32)
        p0b = jax.lax.dot_general(wq[64:128, :], k0, dn0, preferred_element_type=f32)
        p1a = jax.lax.dot_general(wq[0:64, :], k1, dn0, preferred_element_type=f32)
        p1b = jax.lax.dot_general(wq[64:128, :], k1, dn0, preferred_element_type=f32)
        h0 = h0_ref[...]
        h1 = h1_ref[...]
        a0a = jnp.dot(h0, p0a, preferred_element_type=f32)
        a0b = jnp.dot(h0, p0b, preferred_element_type=f32)
        a1a = jnp.dot(h1, p1a, preferred_element_type=f32)
        a1b = jnp.dot(h1, p1b, preferred_element_type=f32)

        col = jax.lax.broadcasted_iota(jnp.int32, (8, 64), 1)
        row = jax.lax.broadcasted_iota(jnp.int32, (8, 64), 0)
        e1 = (col // 8 == row).astype(f32)
        e2 = (col % 8 == row).astype(f32)

        def _masked_softmax_top8(s64):
            cur = s64
            m0 = jnp.max(cur, axis=1, keepdims=True)
            m = m0
            for _ in range(_TOPK - 1):
                cur = jnp.where(cur >= m, -jnp.inf, cur)
                m = jnp.max(cur, axis=1, keepdims=True)
            p = jnp.where(s64 >= m, jnp.exp(s64 - m0), 0.0)
            return p / jnp.sum(p, axis=1, keepdims=True)

        s64e = (jnp.dot(a0a, e1, preferred_element_type=f32)
                + jnp.dot(a1a, e2, preferred_element_type=f32))
        s64o = (jnp.dot(a0b, e1, preferred_element_type=f32)
                + jnp.dot(a1b, e2, preferred_element_type=f32))
        pe = _masked_softmax_top8(s64e)
        po = _masked_softmax_top8(s64o)

        rr = jax.lax.broadcasted_iota(jnp.int32, (_TN, _TN // 2), 0)
        cc = jax.lax.broadcasted_iota(jnp.int32, (_TN, _TN // 2), 1)
        ea = (rr == 2 * cc).astype(f32)
        eb = (rr == 2 * cc + 1).astype(f32)
        p = (jnp.dot(ea, pe, preferred_element_type=f32)
             + jnp.dot(eb, po, preferred_element_type=f32))

        L = jax.lax.dot_general(hb, down_ref[...].astype(jnp.bfloat16),
                                (((1,), (1,)), ((), ())),
                                preferred_element_type=f32)
        w64 = L * jax.nn.sigmoid(L) * p
        out_ref[...] = jnp.dot(w64, up_ref[...], preferred_element_type=f32)

        g = jax.lax.dot_general(hb, wg_ref[...], dnT, preferred_element_type=f32)
        u = jax.lax.dot_general(hb, wu_ref[...], dnT, preferred_element_type=f32)
        a_ref[pl.ds(0, 1)] = (g * jax.nn.sigmoid(g) * u).astype(jnp.bfloat16)[None]

    @pl.when(jnp.logical_and(k > 0, k < kt))
    def _steady():
        hb = hflat_ref[...]
        rd = jax.lax.rem(k + 1, 2)
        wr = jax.lax.rem(k, 2)
        ap = a_ref[pl.ds(rd, 1)][0]
        out_ref[...] += jax.lax.dot_general(
            ap, wd_ref[...], dnT, preferred_element_type=f32)
        g = jax.lax.dot_general(hb, wg_ref[...], dnT, preferred_element_type=f32)
        u = jax.lax.dot_general(hb, wu_ref[...], dnT, preferred_element_type=f32)
        a_ref[pl.ds(wr, 1)] = (g * jax.nn.sigmoid(g) * u).astype(jnp.bfloat16)[None]

    @pl.when(k == kt)
    def _drain():
        out_ref[...] += jax.lax.dot_general(
            a_ref[pl.ds((kt - 1) % 2, 1)][0], wd_ref[...], dnT,
            preferred_element_type=f32)


def kernel(hidden_states, Wq, keys, down_embed, up_embed, Wg, Wu, Wd):
    b, s, h = hidden_states.shape
    N = b * s
    hflat_f = hidden_states.reshape(N, h)
    keys2 = keys.reshape(2 * (_RET // 2), _NK)

    rp = 256
    nchunk = _INTER_PAD // rp
    wg_p, wu_p, wd_p, hflat = pl.pallas_call(
        _repack_kernel,
        grid=(nchunk,),
        in_specs=[
            pl.BlockSpec((rp, h), lambda i: (i, 0)),
            pl.BlockSpec((rp, h), lambda i: (i, 0)),
            pl.BlockSpec((h, rp), lambda i: (0, i)),
            pl.BlockSpec((rp, h), lambda i: (jnp.minimum(i, N // rp - 1), 0)),
        ],
        out_specs=[
            pl.BlockSpec((rp, h), lambda i: (i, 0)),
            pl.BlockSpec((rp, h), lambda i: (i, 0)),
            pl.BlockSpec((h, rp), lambda i: (0, i)),
            pl.BlockSpec((rp, h), lambda i: (jnp.minimum(i, N // rp - 1), 0)),
        ],
        out_shape=[
            jax.ShapeDtypeStruct((_INTER_PAD, h), jnp.bfloat16),
            jax.ShapeDtypeStruct((_INTER_PAD, h), jnp.bfloat16),
            jax.ShapeDtypeStruct((h, _INTER_PAD), jnp.bfloat16),
            jax.ShapeDtypeStruct((N, h), jnp.bfloat16),
        ],
    )(Wg, Wu, Wd, hflat_f)

    nt = N // _TN
    kt = _INTER_PAD // _TK
    out = pl.pallas_call(
        _moe_mlp_kernel,
        grid=(nt, kt + 1),
        in_specs=[
            pl.BlockSpec((_TN // 2, h), lambda n, k: (n, 0)),
            pl.BlockSpec((_TN // 2, h), lambda n, k: (n + _S // (_TN // 2), 0)),
            pl.BlockSpec((_TN, h), lambda n, k: (n, 0)),
            pl.BlockSpec((_RET, h), lambda n, k: (0, 0)),
            pl.BlockSpec((2 * (_RET // 2), _NK), lambda n, k: (0, 0)),
            pl.BlockSpec((_NE, h), lambda n, k: (0, 0)),
            pl.BlockSpec((_NE, h), lambda n, k: (0, 0)),
            pl.BlockSpec((_TK, h), lambda n, k: (jnp.minimum(k, kt - 1), 0)),
            pl.BlockSpec((_TK, h), lambda n, k: (jnp.minimum(k, kt - 1), 0)),
            pl.BlockSpec((h, _TK), lambda n, k: (0, jnp.maximum(k - 1, 0))),
        ],
        out_specs=pl.BlockSpec((_TN, h), lambda n, k: (n, 0)),
        out_shape=jax.ShapeDtypeStruct((N, h), jnp.float32),
        scratch_shapes=[pltpu.VMEM((2, _TN, _TK), jnp.bfloat16)],
        compiler_params=pltpu.CompilerParams(
            dimension_semantics=("arbitrary", "arbitrary"),
        ),
    )(hflat, hflat, hflat, Wq, keys2,
      down_embed, up_embed, wg_p, wu_p, wd_p)

    return out.reshape(b, s, h)
